# baseline (speedup 1.0000x reference)
import functools

import jax
import jax.numpy as jnp
from jax import lax
from jax.experimental import pallas as pl
from jax.experimental.pallas import tpu as pltpu
from jax.experimental.pallas import tpu_sc as plsc

N_USERS = 20000
N_ITEMS = 30000
NN = N_USERS + N_ITEMS
D = 64
E = 800000
NC, NS, LN = 2, 16, 16

HALF = NN // 2
GAP = 88
HCAP = HALF + GAP
TROWS = HCAP // NS
NPAD = NC * HCAP

B = 1024
EPAD = 819200
C32 = EPAD // 32
NB32 = C32 // B
FB = B + LN
CAP = 26880

RDLEN = 51200
RSL = RDLEN // NS
ZR = 56
SB = 128

_f32 = jnp.float32
_i32 = jnp.int32


def _c(v):
    return jnp.int32(v)


def _rsqrt16(x):
    i = lax.bitcast_convert_type(x, _i32)
    y = lax.bitcast_convert_type(_c(0x5F3759DF) - (i >> _c(1)), _f32)
    for _ in range(3):
        y = y * (1.5 - 0.5 * x * y * y)
    return y


def _prep_body(src2d, dst2d, msk_h, val_h,
               srcc_h, dstc_h, nrmc_h, cnts_h,
               srcc2, dstc2, mskc, valc, tmp_v, rd_full,
               bs0, bd0, bn0, bs1, bd1, bn1, cvec, counts_sh, sem):
    c = lax.axis_index("c")
    s = lax.axis_index("s")

    zv = jnp.zeros((LN,), _f32)

    def zero_tmp(i, _):
        tmp_v[pl.ds(i * _c(LN), LN)] = zv
        return 0
    lax.fori_loop(_c(0), _c(RSL // LN), zero_tmp, 0)
    pltpu.sync_copy(tmp_v, counts_sh.at[pl.ds(s * _c(RSL), RSL)])
    plsc.subcore_barrier()

    def count_blk(blk, _):
        off = s * _c(2 * C32) + blk * _c(B)
        row0 = s * _c(2 * C32 // 128) + blk * _c(B // 128)
        pltpu.sync_copy(dst2d.at[pl.ds(row0, B // 128)], dstc2)
        pltpu.sync_copy(val_h.at[pl.ds(off, B)], valc)
        descs = []
        for j in range(B // 128):
            descs.append(pltpu.async_copy(
                valc.at[pl.ds(j * 128, 128)],
                counts_sh.at[dstc2.at[_c(j)]], sem, add=True))
        for dsc in descs:
            dsc.wait()
        return 0
    lax.fori_loop(_c(0), _c(2 * NB32), count_blk, 0)
    plsc.subcore_barrier()

    pltpu.sync_copy(counts_sh.at[pl.ds(s * _c(RSL), RSL)], tmp_v)

    def rsq(i, _):
        deg = jnp.maximum(tmp_v[pl.ds(i * _c(LN), LN)], 1.0)
        tmp_v[pl.ds(i * _c(LN), LN)] = _rsqrt16(deg)
        return 0
    lax.fori_loop(_c(0), _c(RSL // LN), rsq, 0)
    pltpu.sync_copy(tmp_v, counts_sh.at[pl.ds(s * _c(RSL), RSL)])
    plsc.subcore_barrier()

    pltpu.sync_copy(counts_sh, rd_full)

    wid = s * _c(NC) + c
    it = jnp.arange(LN, dtype=_i32)
    nsrc = (it * _c(641)) & _c(16383)
    ndst = (it * _c(389)) & _c(8191)

    def comp_blk(blk, carry):
        h0, h1 = carry
        off = wid * _c(C32) + blk * _c(B)
        row0 = wid * _c(C32 // 128) + blk * _c(B // 128)
        pltpu.sync_copy(src2d.at[pl.ds(row0, B // 128)], srcc2)
        pltpu.sync_copy(dst2d.at[pl.ds(row0, B // 128)], dstc2)
        pltpu.sync_copy(msk_h.at[pl.ds(off, B)], mskc)

        def vreg(i, oo):
            o0, o1 = oo
            j = i >> _c(3)
            kc = (i & _c(7)) * _c(LN)
            sv = srcc2[j, pl.ds(kc, LN)]
            dv = dstc2[j, pl.ds(kc, LN)]
            m = mskc[pl.ds(i * _c(LN), LN)]
            rs = plsc.load_gather(rd_full, [sv])
            rdd = plsc.load_gather(rd_full, [dv])
            nrm = m * rs * rdd
            g = jnp.where(sv >= _c(HALF), sv + _c(GAP), sv)
            m0 = dv < _c(HALF)
            plsc.store_compressed(bs0.at[pl.ds(o0, LN)], g, mask=m0)
            plsc.store_compressed(bd0.at[pl.ds(o0, LN)], dv, mask=m0)
            plsc.store_compressed(bn0.at[pl.ds(o0, LN)], nrm, mask=m0)
            m1 = jnp.logical_not(m0)
            plsc.store_compressed(bs1.at[pl.ds(o1, LN)], g, mask=m1)
            plsc.store_compressed(bd1.at[pl.ds(o1, LN)], dv - _c(HALF),
                                  mask=m1)
            plsc.store_compressed(bn1.at[pl.ds(o1, LN)], nrm, mask=m1)
            c0 = plsc.all_reduce_population_count(m0)[0]
            return (o0 + c0, o1 + (_c(LN) - c0))
        o0, o1 = lax.fori_loop(_c(0), _c(B // LN), vreg, (_c(0), _c(0)))

        bs0[pl.ds(o0, LN)] = nsrc
        bd0[pl.ds(o0, LN)] = ndst
        bn0[pl.ds(o0, LN)] = jnp.zeros((LN,), _f32)
        bs1[pl.ds(o1, LN)] = nsrc
        bd1[pl.ds(o1, LN)] = ndst
        bn1[pl.ds(o1, LN)] = jnp.zeros((LN,), _f32)
        ha = pl.multiple_of(h0, 8)
        hb = pl.multiple_of(h1, 8)
        pltpu.sync_copy(bs0, srcc_h.at[_c(0), wid, pl.ds(ha, FB)])
        pltpu.sync_copy(bd0, dstc_h.at[_c(0), wid, pl.ds(ha, FB)])
        pltpu.sync_copy(bn0, nrmc_h.at[_c(0), wid, pl.ds(ha, FB)])
        pltpu.sync_copy(bs1, srcc_h.at[_c(1), wid, pl.ds(hb, FB)])
        pltpu.sync_copy(bd1, dstc_h.at[_c(1), wid, pl.ds(hb, FB)])
        pltpu.sync_copy(bn1, nrmc_h.at[_c(1), wid, pl.ds(hb, FB)])
        return (h0 + ((o0 + _c(7)) & _c(-8)), h1 + ((o1 + _c(7)) & _c(-8)))
    h0, h1 = lax.fori_loop(_c(0), _c(NB32), comp_blk, (_c(0), _c(0)))

    def nfill(t, _):
        o = t * _c(LN)
        bs0[pl.ds(o, LN)] = nsrc
        bd0[pl.ds(o, LN)] = ndst
        bn0[pl.ds(o, LN)] = jnp.zeros((LN,), _f32)
        return 0
    lax.fori_loop(_c(0), _c(FB // LN), nfill, 0)
    ha = pl.multiple_of(h0, 8)
    hb = pl.multiple_of(h1, 8)
    pltpu.sync_copy(bs0, srcc_h.at[_c(0), wid, pl.ds(ha, FB)])
    pltpu.sync_copy(bd0, dstc_h.at[_c(0), wid, pl.ds(ha, FB)])
    pltpu.sync_copy(bn0, nrmc_h.at[_c(0), wid, pl.ds(ha, FB)])
    pltpu.sync_copy(bs0, srcc_h.at[_c(1), wid, pl.ds(hb, FB)])
    pltpu.sync_copy(bd0, dstc_h.at[_c(1), wid, pl.ds(hb, FB)])
    pltpu.sync_copy(bn0, nrmc_h.at[_c(1), wid, pl.ds(hb, FB)])

    crow = jnp.where(it == _c(0), h0, jnp.where(it == _c(1), h1, _c(0)))
    cvec[pl.ds(_c(0), LN)] = crow
    pltpu.sync_copy(cvec, cnts_h.at[wid])


def _edge_pass(c, s, srcc4, dstc4, nrmc3, cnts_h, emb_h,
               srcc2, dstc2, nrmc, rows_v, cv, acc_sh, sem,
               sg0, sg1, ss0, ss1):
    zv = jnp.zeros((LN,), _f32)

    def zb(i, _):
        rows_v[i >> _c(2), pl.ds((i & _c(3)) * _c(LN), LN)] = zv
        return 0
    lax.fori_loop(_c(0), _c(ZR * D // LN), zb, 0)
    for k in range(TROWS // ZR):
        pltpu.sync_copy(rows_v.at[pl.ds(0, ZR)],
                        acc_sh.at[pl.ds(s * _c(TROWS) + _c(k * ZR), ZR)])
    plsc.subcore_barrier()

    pltpu.sync_copy(cnts_h.at[pl.ds(s * _c(2), 2)], cv)
    v0 = cv[_c(0), pl.ds(0, LN)]
    v1 = cv[_c(1), pl.ds(0, LN)]
    is0 = c == _c(0)
    nn = (jnp.where(is0, v0[0], v0[1]), jnp.where(is0, v1[0], v1[1]))

    for li in range(2):
        lt = s * _c(2) + _c(li)
        nblk = (nn[li] + _c(B - 1)) >> _c(10)

        def blk_fn(blk, _):
            pltpu.sync_copy(srcc4.at[c, lt, pl.ds(blk * _c(B // 128),
                                                  B // 128)], srcc2)
            pltpu.sync_copy(dstc4.at[c, lt, pl.ds(blk * _c(B // 128),
                                                  B // 128)], dstc2)
            pltpu.sync_copy(nrmc3.at[c, lt, pl.ds(blk * _c(B), B)], nrmc)

            row0 = rows_v.at[pl.ds(0, 128)]
            row1 = rows_v.at[pl.ds(128, 128)]

            def g_desc(qq, slot, sm):
                return pltpu.make_async_copy(emb_h.at[srcc2.at[qq]], slot, sm)

            def s_desc(qq, slot, sm):
                return pltpu.make_async_copy(slot, acc_sh.at[dstc2.at[qq]],
                                             sm)

            def scale_slot(base_row, woff):
                def scale(g, _):
                    w16 = nrmc[pl.ds(woff + g * _c(LN), LN)]
                    e0 = _c(base_row) + g * _c(LN)
                    for l in range(LN):
                        e = e0 + _c(l)
                        w = w16[jnp.full((LN,), l, _i32)]
                        for kk in range(D // LN):
                            rows_v[e, pl.ds(kk * LN, LN)] = (
                                rows_v[e, pl.ds(kk * LN, LN)] * w)
                    return 0
                lax.fori_loop(_c(0), _c(128 // LN), scale, 0)

            g_desc(_c(0), row0, sg0).start()

            def pair(p, _):
                q0 = p * _c(2)
                q1 = q0 + _c(1)

                @pl.when(p > _c(0))
                def _wait_s1():
                    s_desc(q1 - _c(2), row1, ss1).wait()
                g_desc(q1, row1, sg1).start()
                g_desc(q0, row0, sg0).wait()
                scale_slot(0, q0 * _c(128))
                s_desc(q0, row0, ss0).start(add=True)
                g_desc(q1, row1, sg1).wait()
                scale_slot(128, q1 * _c(128))
                s_desc(q1, row1, ss1).start(add=True)

                @pl.when(p < _c(B // SB // 2 - 1))
                def _next_g0():
                    s_desc(q0, row0, ss0).wait()
                    g_desc(q0 + _c(2), row0, sg0).start()
                return 0
            lax.fori_loop(_c(0), _c(B // SB // 2), pair, 0)
            s_desc(_c(B // SB - 2), row0, ss0).wait()
            s_desc(_c(B // SB - 1), row1, ss1).wait()
            return 0
        lax.fori_loop(_c(0), nblk, blk_fn, 0)
    plsc.subcore_barrier()


def _layer_body(emb_h, srcc4, dstc4, nrmc3, cnts_h, out_h,
                srcc2, dstc2, nrmc, rows_v, cv, acc_sh, sem,
                sg0, sg1, ss0, ss1):
    c = lax.axis_index("c")
    s = lax.axis_index("s")
    _edge_pass(c, s, srcc4, dstc4, nrmc3, cnts_h, emb_h,
               srcc2, dstc2, nrmc, rows_v, cv, acc_sh, sem,
               sg0, sg1, ss0, ss1)
    pltpu.sync_copy(acc_sh.at[pl.ds(s * _c(TROWS), TROWS)],
                    out_h.at[pl.ds(c * _c(HCAP) + s * _c(TROWS), TROWS)])


def _final_body(emb_h, srcc4, dstc4, nrmc3, cnts_h, e0p_h, e1p_h, out_h,
                srcc2, dstc2, nrmc, rows_v, cv, acc_sh, sem,
                sg0, sg1, ss0, ss1):
    c = lax.axis_index("c")
    s = lax.axis_index("s")
    _edge_pass(c, s, srcc4, dstc4, nrmc3, cnts_h, emb_h,
               srcc2, dstc2, nrmc, rows_v, cv, acc_sh, sem,
               sg0, sg1, ss0, ss1)

    start_pad = c * _c(HCAP) + s * _c(TROWS)
    is_edge = s == _c(NS - 1)

    def mean_blk(k, _):
        boff = k * _c(ZR)
        boff = jnp.where(is_edge, jnp.minimum(boff, _c(TROWS - GAP - ZR)),
                         boff)
        row_pad = start_pad + boff
        out_row = row_pad - _c(GAP) * c
        d0 = pltpu.async_copy(acc_sh.at[pl.ds(s * _c(TROWS) + boff, ZR)],
                              rows_v.at[pl.ds(0, ZR)], sg0)
        d1 = pltpu.async_copy(e0p_h.at[pl.ds(row_pad, ZR)],
                              rows_v.at[pl.ds(ZR, ZR)], sg1)
        d2 = pltpu.async_copy(e1p_h.at[pl.ds(row_pad, ZR)],
                              rows_v.at[pl.ds(2 * ZR, ZR)], ss0)
        d3 = pltpu.async_copy(emb_h.at[pl.ds(row_pad, ZR)],
                              rows_v.at[pl.ds(3 * ZR, ZR)], ss1)
        for dd in (d0, d1, d2, d3):
            dd.wait()

        def vreg(i, _):
            r = i >> _c(2)
            kc = (i & _c(3)) * _c(LN)
            v = (rows_v[r, pl.ds(kc, LN)]
                 + rows_v[_c(ZR) + r, pl.ds(kc, LN)]
                 + rows_v[_c(2 * ZR) + r, pl.ds(kc, LN)]
                 + rows_v[_c(3 * ZR) + r, pl.ds(kc, LN)])
            rows_v[r, pl.ds(kc, LN)] = v * 0.25
            return 0
        lax.fori_loop(_c(0), _c(ZR * D // LN), vreg, 0)
        pltpu.sync_copy(rows_v.at[pl.ds(0, ZR)],
                        out_h.at[pl.ds(out_row, ZR)])
        return 0
    lax.fori_loop(_c(0), _c(TROWS // ZR), mean_blk, 0)


@functools.lru_cache(maxsize=1)
def _get_calls():
    mesh = plsc.VectorSubcoreMesh(core_axis_name="c", subcore_axis_name="s",
                                  num_cores=NC, num_subcores=NS)
    params = pltpu.CompilerParams(needs_layout_passes=False,
                                  use_tc_tiling_on_sc=False)
    prep_call = pl.kernel(
        _prep_body,
        out_type=(
            jax.ShapeDtypeStruct((NC, 32, CAP), _i32),
            jax.ShapeDtypeStruct((NC, 32, CAP), _i32),
            jax.ShapeDtypeStruct((NC, 32, CAP), _f32),
            jax.ShapeDtypeStruct((32, LN), _i32),
        ),
        mesh=mesh,
        compiler_params=params,
        scratch_types=[
            pltpu.VMEM((8, 128), _i32),
            pltpu.VMEM((8, 128), _i32),
            pltpu.VMEM((B,), _f32),
            pltpu.VMEM((B,), _f32),
            pltpu.VMEM((RSL,), _f32),
            pltpu.VMEM((RDLEN,), _f32),
            pltpu.VMEM((FB,), _i32),
            pltpu.VMEM((FB,), _i32),
            pltpu.VMEM((FB,), _f32),
            pltpu.VMEM((FB,), _i32),
            pltpu.VMEM((FB,), _i32),
            pltpu.VMEM((FB,), _f32),
            pltpu.VMEM((LN,), _i32),
            pltpu.VMEM_SHARED((RDLEN,), _f32),
            pltpu.SemaphoreType.DMA,
        ],
    )
    layer_scratch = [
        pltpu.VMEM((8, 128), _i32),
        pltpu.VMEM((8, 128), _i32),
        pltpu.VMEM((B,), _f32),
        pltpu.VMEM((2 * SB, D), _f32),
        pltpu.VMEM((2, LN), _i32),
    ]
    layer_call = pl.kernel(
        _layer_body,
        out_type=jax.ShapeDtypeStruct((NPAD, D), _f32),
        mesh=mesh,
        compiler_params=params,
        scratch_types=layer_scratch + [
            pltpu.VMEM_SHARED((HCAP, D), _f32),
            pltpu.SemaphoreType.DMA,
            pltpu.SemaphoreType.DMA,
            pltpu.SemaphoreType.DMA,
            pltpu.SemaphoreType.DMA,
            pltpu.SemaphoreType.DMA,
        ],
    )
    final_call = pl.kernel(
        _final_body,
        out_type=jax.ShapeDtypeStruct((NN, D), _f32),
        mesh=mesh,
        compiler_params=params,
        scratch_types=layer_scratch + [
            pltpu.VMEM_SHARED((HCAP, D), _f32),
            pltpu.SemaphoreType.DMA,
            pltpu.SemaphoreType.DMA,
            pltpu.SemaphoreType.DMA,
            pltpu.SemaphoreType.DMA,
            pltpu.SemaphoreType.DMA,
        ],
    )
    return prep_call, layer_call, final_call


def kernel(user_emb, item_emb, edge_index, mask):
    src = edge_index[0].astype(_i32)
    dst = edge_index[1].astype(_i32)
    mask = mask.astype(_f32)

    padn = EPAD - E
    padidx = (jnp.arange(padn, dtype=_i32) * 37) % NN
    srcp = jnp.concatenate([src, padidx])
    dstp = jnp.concatenate([dst, padidx])
    maskp = jnp.concatenate([mask, jnp.zeros((padn,), _f32)])
    validp = jnp.concatenate([jnp.ones((E,), _f32), jnp.zeros((padn,), _f32)])
    src2d = srcp.reshape(EPAD // 128, 128)
    dst2d = dstp.reshape(EPAD // 128, 128)

    gap = jnp.zeros((GAP, D), _f32)
    e0p = jnp.concatenate([user_emb, item_emb[:HALF - N_USERS], gap,
                           item_emb[HALF - N_USERS:], gap])

    prep_call, layer_call, final_call = _get_calls()
    srcc, dstc, nrmc, cnts = prep_call(src2d, dst2d, maskp, validp)
    srcc4 = srcc.reshape(NC, 32, CAP // 128, 128)
    dstc4 = dstc.reshape(NC, 32, CAP // 128, 128)
    e1p = layer_call(e0p, srcc4, dstc4, nrmc, cnts)
    e2p = layer_call(e1p, srcc4, dstc4, nrmc, cnts)
    out = final_call(e2p, srcc4, dstc4, nrmc, cnts, e0p, e1p)
    return out


# speedup vs baseline: 1.7572x; 1.7572x over previous
import functools

import jax
import jax.numpy as jnp
from jax import lax
from jax.experimental import pallas as pl
from jax.experimental.pallas import tpu as pltpu
from jax.experimental.pallas import tpu_sc as plsc

N_USERS = 20000
N_ITEMS = 30000
NN = N_USERS + N_ITEMS
D = 64
E = 800000
NC, NS, LN = 2, 16, 16

HALF = NN // 2
GAP = 88
HCAP = HALF + GAP
TROWS = HCAP // NS
NPAD = NC * HCAP

B = 1024
EPAD = 819200
C32 = EPAD // 32
NB32 = C32 // B
FB = B + LN
CAP = 26880

RDLEN = 51200
RSL = RDLEN // NS
ZR = 56
SB = 128

_f32 = jnp.float32
_i32 = jnp.int32


def _c(v):
    return jnp.int32(v)


def _rsqrt16(x):
    i = lax.bitcast_convert_type(x, _i32)
    y = lax.bitcast_convert_type(_c(0x5F3759DF) - (i >> _c(1)), _f32)
    for _ in range(3):
        y = y * (1.5 - 0.5 * x * y * y)
    return y


def _prep_body(src2d, dst2d, msk_h, val_h,
               srcc_h, dstc_h, nrmc_h, cnts_h,
               srcc2, dstc2, mskc, valc, tmp_v, rd_full,
               bs0, bd0, bn0, bs1, bd1, bn1, cvec, counts_sh, sem):
    c = lax.axis_index("c")
    s = lax.axis_index("s")

    zv = jnp.zeros((LN,), _f32)

    def zero_tmp(i, _):
        tmp_v[pl.ds(i * _c(LN), LN)] = zv
        return 0
    lax.fori_loop(_c(0), _c(RSL // LN), zero_tmp, 0)
    pltpu.sync_copy(tmp_v, counts_sh.at[pl.ds(s * _c(RSL), RSL)])
    plsc.subcore_barrier()

    def count_blk(blk, _):
        off = s * _c(2 * C32) + blk * _c(B)
        row0 = s * _c(2 * C32 // 128) + blk * _c(B // 128)
        pltpu.sync_copy(dst2d.at[pl.ds(row0, B // 128)], dstc2)
        pltpu.sync_copy(val_h.at[pl.ds(off, B)], valc)
        descs = []
        for j in range(B // 128):
            descs.append(pltpu.async_copy(
                valc.at[pl.ds(j * 128, 128)],
                counts_sh.at[dstc2.at[_c(j)]], sem, add=True))
        for dsc in descs:
            dsc.wait()
        return 0
    lax.fori_loop(_c(0), _c(2 * NB32), count_blk, 0)
    plsc.subcore_barrier()

    pltpu.sync_copy(counts_sh.at[pl.ds(s * _c(RSL), RSL)], tmp_v)

    def rsq(i, _):
        deg = jnp.maximum(tmp_v[pl.ds(i * _c(LN), LN)], 1.0)
        tmp_v[pl.ds(i * _c(LN), LN)] = _rsqrt16(deg)
        return 0
    lax.fori_loop(_c(0), _c(RSL // LN), rsq, 0)
    pltpu.sync_copy(tmp_v, counts_sh.at[pl.ds(s * _c(RSL), RSL)])
    plsc.subcore_barrier()

    pltpu.sync_copy(counts_sh, rd_full)

    wid = s * _c(NC) + c
    it = jnp.arange(LN, dtype=_i32)
    nsrc = (it * _c(641)) & _c(16383)
    ndst = (it * _c(389)) & _c(8191)

    def comp_blk(blk, carry):
        h0, h1 = carry
        off = wid * _c(C32) + blk * _c(B)
        row0 = wid * _c(C32 // 128) + blk * _c(B // 128)
        pltpu.sync_copy(src2d.at[pl.ds(row0, B // 128)], srcc2)
        pltpu.sync_copy(dst2d.at[pl.ds(row0, B // 128)], dstc2)
        pltpu.sync_copy(msk_h.at[pl.ds(off, B)], mskc)

        def vreg(i, oo):
            o0, o1 = oo
            j = i >> _c(3)
            kc = (i & _c(7)) * _c(LN)
            sv = srcc2[j, pl.ds(kc, LN)]
            dv = dstc2[j, pl.ds(kc, LN)]
            m = mskc[pl.ds(i * _c(LN), LN)]
            rs = plsc.load_gather(rd_full, [sv])
            rdd = plsc.load_gather(rd_full, [dv])
            nrm = m * rs * rdd
            g = jnp.where(sv >= _c(HALF), sv + _c(GAP), sv)
            m0 = dv < _c(HALF)
            plsc.store_compressed(bs0.at[pl.ds(o0, LN)], g, mask=m0)
            plsc.store_compressed(bd0.at[pl.ds(o0, LN)], dv, mask=m0)
            plsc.store_compressed(bn0.at[pl.ds(o0, LN)], nrm, mask=m0)
            m1 = jnp.logical_not(m0)
            plsc.store_compressed(bs1.at[pl.ds(o1, LN)], g, mask=m1)
            plsc.store_compressed(bd1.at[pl.ds(o1, LN)], dv - _c(HALF),
                                  mask=m1)
            plsc.store_compressed(bn1.at[pl.ds(o1, LN)], nrm, mask=m1)
            c0 = plsc.all_reduce_population_count(m0)[0]
            return (o0 + c0, o1 + (_c(LN) - c0))
        o0, o1 = lax.fori_loop(_c(0), _c(B // LN), vreg, (_c(0), _c(0)))

        bs0[pl.ds(o0, LN)] = nsrc
        bd0[pl.ds(o0, LN)] = ndst
        bn0[pl.ds(o0, LN)] = jnp.zeros((LN,), _f32)
        bs1[pl.ds(o1, LN)] = nsrc
        bd1[pl.ds(o1, LN)] = ndst
        bn1[pl.ds(o1, LN)] = jnp.zeros((LN,), _f32)
        ha = pl.multiple_of(h0, 8)
        hb = pl.multiple_of(h1, 8)
        pltpu.sync_copy(bs0, srcc_h.at[_c(0), wid, pl.ds(ha, FB)])
        pltpu.sync_copy(bd0, dstc_h.at[_c(0), wid, pl.ds(ha, FB)])
        pltpu.sync_copy(bn0, nrmc_h.at[_c(0), wid, pl.ds(ha, FB)])
        pltpu.sync_copy(bs1, srcc_h.at[_c(1), wid, pl.ds(hb, FB)])
        pltpu.sync_copy(bd1, dstc_h.at[_c(1), wid, pl.ds(hb, FB)])
        pltpu.sync_copy(bn1, nrmc_h.at[_c(1), wid, pl.ds(hb, FB)])
        return (h0 + ((o0 + _c(7)) & _c(-8)), h1 + ((o1 + _c(7)) & _c(-8)))
    h0, h1 = lax.fori_loop(_c(0), _c(NB32), comp_blk, (_c(0), _c(0)))

    def nfill(t, _):
        o = t * _c(LN)
        bs0[pl.ds(o, LN)] = nsrc
        bd0[pl.ds(o, LN)] = ndst
        bn0[pl.ds(o, LN)] = jnp.zeros((LN,), _f32)
        return 0
    lax.fori_loop(_c(0), _c(FB // LN), nfill, 0)
    ha = pl.multiple_of(h0, 8)
    hb = pl.multiple_of(h1, 8)
    pltpu.sync_copy(bs0, srcc_h.at[_c(0), wid, pl.ds(ha, FB)])
    pltpu.sync_copy(bd0, dstc_h.at[_c(0), wid, pl.ds(ha, FB)])
    pltpu.sync_copy(bn0, nrmc_h.at[_c(0), wid, pl.ds(ha, FB)])
    pltpu.sync_copy(bs0, srcc_h.at[_c(1), wid, pl.ds(hb, FB)])
    pltpu.sync_copy(bd0, dstc_h.at[_c(1), wid, pl.ds(hb, FB)])
    pltpu.sync_copy(bn0, nrmc_h.at[_c(1), wid, pl.ds(hb, FB)])

    crow = jnp.where(it == _c(0), h0, jnp.where(it == _c(1), h1, _c(0)))
    cvec[pl.ds(_c(0), LN)] = crow
    pltpu.sync_copy(cvec, cnts_h.at[wid])


def _edge_pass(c, s, srcc4, dstc4, nrmc3, cnts_h, emb_h,
               srcc2, dstc2, nrmc, rows_v, cv, acc_sh, sem,
               sg0, sg1, ss0, ss1):
    zv = jnp.zeros((LN,), _f32)

    def zb(i, _):
        rows_v[i >> _c(2), pl.ds((i & _c(3)) * _c(LN), LN)] = zv
        return 0
    lax.fori_loop(_c(0), _c(ZR * D // LN), zb, 0)
    for k in range(TROWS // ZR):
        pltpu.sync_copy(rows_v.at[pl.ds(0, ZR)],
                        acc_sh.at[pl.ds(s * _c(TROWS) + _c(k * ZR), ZR)])
    plsc.subcore_barrier()

    pltpu.sync_copy(cnts_h.at[pl.ds(s * _c(2), 2)], cv)
    v0 = cv[_c(0), pl.ds(0, LN)]
    v1 = cv[_c(1), pl.ds(0, LN)]
    is0 = c == _c(0)
    nn = (jnp.where(is0, v0[0], v0[1]), jnp.where(is0, v1[0], v1[1]))

    for li in range(2):
        lt = s * _c(2) + _c(li)
        nblk = (nn[li] + _c(B - 1)) >> _c(10)

        def blk_fn(blk, _):
            pltpu.sync_copy(srcc4.at[c, lt, pl.ds(blk * _c(B // 128),
                                                  B // 128)], srcc2)
            pltpu.sync_copy(dstc4.at[c, lt, pl.ds(blk * _c(B // 128),
                                                  B // 128)], dstc2)
            pltpu.sync_copy(nrmc3.at[c, lt, pl.ds(blk * _c(B), B)], nrmc)

            row0 = rows_v.at[pl.ds(0, 128)]
            row1 = rows_v.at[pl.ds(128, 128)]

            def g_desc(qq, slot, sm):
                return pltpu.make_async_copy(emb_h.at[srcc2.at[qq]], slot, sm)

            def s_desc(qq, slot, sm):
                return pltpu.make_async_copy(slot, acc_sh.at[dstc2.at[qq]],
                                             sm)

            def scale_slot(base_row, woff):
                @plsc.parallel_loop(_c(0), _c(128 // LN), _c(1), unroll=2)
                def scale(g):
                    w16 = nrmc[pl.ds(woff + g * _c(LN), LN)]
                    e0 = _c(base_row) + g * _c(LN)
                    for l in range(LN):
                        e = e0 + _c(l)
                        w = w16[jnp.full((LN,), l, _i32)]
                        for kk in range(D // LN):
                            rows_v[e, pl.ds(kk * LN, LN)] = (
                                rows_v[e, pl.ds(kk * LN, LN)] * w)

            g_desc(_c(0), row0, sg0).start()

            def pair(p, _):
                q0 = p * _c(2)
                q1 = q0 + _c(1)

                @pl.when(p > _c(0))
                def _wait_s1():
                    s_desc(q1 - _c(2), row1, ss1).wait()
                g_desc(q1, row1, sg1).start()
                g_desc(q0, row0, sg0).wait()
                scale_slot(0, q0 * _c(128))
                s_desc(q0, row0, ss0).start(add=True)
                g_desc(q1, row1, sg1).wait()
                scale_slot(128, q1 * _c(128))
                s_desc(q1, row1, ss1).start(add=True)

                @pl.when(p < _c(B // SB // 2 - 1))
                def _next_g0():
                    s_desc(q0, row0, ss0).wait()
                    g_desc(q0 + _c(2), row0, sg0).start()
                return 0
            lax.fori_loop(_c(0), _c(B // SB // 2), pair, 0)
            s_desc(_c(B // SB - 2), row0, ss0).wait()
            s_desc(_c(B // SB - 1), row1, ss1).wait()
            return 0
        lax.fori_loop(_c(0), nblk, blk_fn, 0)
    plsc.subcore_barrier()


def _layer_body(emb_h, srcc4, dstc4, nrmc3, cnts_h, out_h,
                srcc2, dstc2, nrmc, rows_v, cv, acc_sh, sem,
                sg0, sg1, ss0, ss1):
    c = lax.axis_index("c")
    s = lax.axis_index("s")
    _edge_pass(c, s, srcc4, dstc4, nrmc3, cnts_h, emb_h,
               srcc2, dstc2, nrmc, rows_v, cv, acc_sh, sem,
               sg0, sg1, ss0, ss1)
    pltpu.sync_copy(acc_sh.at[pl.ds(s * _c(TROWS), TROWS)],
                    out_h.at[pl.ds(c * _c(HCAP) + s * _c(TROWS), TROWS)])


def _final_body(emb_h, srcc4, dstc4, nrmc3, cnts_h, e0p_h, e1p_h, out_h,
                srcc2, dstc2, nrmc, rows_v, cv, acc_sh, sem,
                sg0, sg1, ss0, ss1):
    c = lax.axis_index("c")
    s = lax.axis_index("s")
    _edge_pass(c, s, srcc4, dstc4, nrmc3, cnts_h, emb_h,
               srcc2, dstc2, nrmc, rows_v, cv, acc_sh, sem,
               sg0, sg1, ss0, ss1)

    start_pad = c * _c(HCAP) + s * _c(TROWS)
    is_edge = s == _c(NS - 1)

    def mean_blk(k, _):
        boff = k * _c(ZR)
        boff = jnp.where(is_edge, jnp.minimum(boff, _c(TROWS - GAP - ZR)),
                         boff)
        row_pad = start_pad + boff
        out_row = row_pad - _c(GAP) * c
        d0 = pltpu.async_copy(acc_sh.at[pl.ds(s * _c(TROWS) + boff, ZR)],
                              rows_v.at[pl.ds(0, ZR)], sg0)
        d1 = pltpu.async_copy(e0p_h.at[pl.ds(row_pad, ZR)],
                              rows_v.at[pl.ds(ZR, ZR)], sg1)
        d2 = pltpu.async_copy(e1p_h.at[pl.ds(row_pad, ZR)],
                              rows_v.at[pl.ds(2 * ZR, ZR)], ss0)
        d3 = pltpu.async_copy(emb_h.at[pl.ds(row_pad, ZR)],
                              rows_v.at[pl.ds(3 * ZR, ZR)], ss1)
        for dd in (d0, d1, d2, d3):
            dd.wait()

        def vreg(i, _):
            r = i >> _c(2)
            kc = (i & _c(3)) * _c(LN)
            v = (rows_v[r, pl.ds(kc, LN)]
                 + rows_v[_c(ZR) + r, pl.ds(kc, LN)]
                 + rows_v[_c(2 * ZR) + r, pl.ds(kc, LN)]
                 + rows_v[_c(3 * ZR) + r, pl.ds(kc, LN)])
            rows_v[r, pl.ds(kc, LN)] = v * 0.25
            return 0
        lax.fori_loop(_c(0), _c(ZR * D // LN), vreg, 0)
        pltpu.sync_copy(rows_v.at[pl.ds(0, ZR)],
                        out_h.at[pl.ds(out_row, ZR)])
        return 0
    lax.fori_loop(_c(0), _c(TROWS // ZR), mean_blk, 0)


@functools.lru_cache(maxsize=1)
def _get_calls():
    mesh = plsc.VectorSubcoreMesh(core_axis_name="c", subcore_axis_name="s",
                                  num_cores=NC, num_subcores=NS)
    params = pltpu.CompilerParams(needs_layout_passes=False,
                                  use_tc_tiling_on_sc=False)
    prep_call = pl.kernel(
        _prep_body,
        out_type=(
            jax.ShapeDtypeStruct((NC, 32, CAP), _i32),
            jax.ShapeDtypeStruct((NC, 32, CAP), _i32),
            jax.ShapeDtypeStruct((NC, 32, CAP), _f32),
            jax.ShapeDtypeStruct((32, LN), _i32),
        ),
        mesh=mesh,
        compiler_params=params,
        scratch_types=[
            pltpu.VMEM((8, 128), _i32),
            pltpu.VMEM((8, 128), _i32),
            pltpu.VMEM((B,), _f32),
            pltpu.VMEM((B,), _f32),
            pltpu.VMEM((RSL,), _f32),
            pltpu.VMEM((RDLEN,), _f32),
            pltpu.VMEM((FB,), _i32),
            pltpu.VMEM((FB,), _i32),
            pltpu.VMEM((FB,), _f32),
            pltpu.VMEM((FB,), _i32),
            pltpu.VMEM((FB,), _i32),
            pltpu.VMEM((FB,), _f32),
            pltpu.VMEM((LN,), _i32),
            pltpu.VMEM_SHARED((RDLEN,), _f32),
            pltpu.SemaphoreType.DMA,
        ],
    )
    layer_scratch = [
        pltpu.VMEM((8, 128), _i32),
        pltpu.VMEM((8, 128), _i32),
        pltpu.VMEM((B,), _f32),
        pltpu.VMEM((2 * SB, D), _f32),
        pltpu.VMEM((2, LN), _i32),
    ]
    layer_call = pl.kernel(
        _layer_body,
        out_type=jax.ShapeDtypeStruct((NPAD, D), _f32),
        mesh=mesh,
        compiler_params=params,
        scratch_types=layer_scratch + [
            pltpu.VMEM_SHARED((HCAP, D), _f32),
            pltpu.SemaphoreType.DMA,
            pltpu.SemaphoreType.DMA,
            pltpu.SemaphoreType.DMA,
            pltpu.SemaphoreType.DMA,
            pltpu.SemaphoreType.DMA,
        ],
    )
    final_call = pl.kernel(
        _final_body,
        out_type=jax.ShapeDtypeStruct((NN, D), _f32),
        mesh=mesh,
        compiler_params=params,
        scratch_types=layer_scratch + [
            pltpu.VMEM_SHARED((HCAP, D), _f32),
            pltpu.SemaphoreType.DMA,
            pltpu.SemaphoreType.DMA,
            pltpu.SemaphoreType.DMA,
            pltpu.SemaphoreType.DMA,
            pltpu.SemaphoreType.DMA,
        ],
    )
    return prep_call, layer_call, final_call


def kernel(user_emb, item_emb, edge_index, mask):
    src = edge_index[0].astype(_i32)
    dst = edge_index[1].astype(_i32)
    mask = mask.astype(_f32)

    padn = EPAD - E
    padidx = (jnp.arange(padn, dtype=_i32) * 37) % NN
    srcp = jnp.concatenate([src, padidx])
    dstp = jnp.concatenate([dst, padidx])
    maskp = jnp.concatenate([mask, jnp.zeros((padn,), _f32)])
    validp = jnp.concatenate([jnp.ones((E,), _f32), jnp.zeros((padn,), _f32)])
    src2d = srcp.reshape(EPAD // 128, 128)
    dst2d = dstp.reshape(EPAD // 128, 128)

    gap = jnp.zeros((GAP, D), _f32)
    e0p = jnp.concatenate([user_emb, item_emb[:HALF - N_USERS], gap,
                           item_emb[HALF - N_USERS:], gap])

    prep_call, layer_call, final_call = _get_calls()
    srcc, dstc, nrmc, cnts = prep_call(src2d, dst2d, maskp, validp)
    srcc4 = srcc.reshape(NC, 32, CAP // 128, 128)
    dstc4 = dstc.reshape(NC, 32, CAP // 128, 128)
    e1p = layer_call(e0p, srcc4, dstc4, nrmc, cnts)
    e2p = layer_call(e1p, srcc4, dstc4, nrmc, cnts)
    out = final_call(e2p, srcc4, dstc4, nrmc, cnts, e0p, e1p)
    return out


# speedup vs baseline: 1.8535x; 1.0548x over previous
import functools

import jax
import jax.numpy as jnp
from jax import lax
from jax.experimental import pallas as pl
from jax.experimental.pallas import tpu as pltpu
from jax.experimental.pallas import tpu_sc as plsc

N_USERS = 20000
N_ITEMS = 30000
NN = N_USERS + N_ITEMS
D = 64
E = 800000
NC, NS, LN = 2, 16, 16

HALF = NN // 2
GAP = 88
HCAP = HALF + GAP
TROWS = HCAP // NS
NPAD = NC * HCAP

B = 1024
EPAD = 819200
C32 = EPAD // 32
NB32 = C32 // B
FB = B + LN
CAP = 26880

RDLEN = 51200
RSL = RDLEN // NS
ZR = 56
SB = 128

_f32 = jnp.float32
_i32 = jnp.int32


def _c(v):
    return jnp.int32(v)


def _rsqrt16(x):
    i = lax.bitcast_convert_type(x, _i32)
    y = lax.bitcast_convert_type(_c(0x5F3759DF) - (i >> _c(1)), _f32)
    for _ in range(3):
        y = y * (1.5 - 0.5 * x * y * y)
    return y


def _prep_body(src2d, dst2d, msk_h, val_h,
               srcc_h, dstc_h, nrmc_h, cnts_h,
               srcc2, dstc2, mskc, valc, tmp_v, rd_full,
               bs0, bd0, bn0, bs1, bd1, bn1, cvec, counts_sh, sem):
    c = lax.axis_index("c")
    s = lax.axis_index("s")

    zv = jnp.zeros((LN,), _f32)

    def zero_tmp(i, _):
        tmp_v[pl.ds(i * _c(LN), LN)] = zv
        return 0
    lax.fori_loop(_c(0), _c(RSL // LN), zero_tmp, 0)
    pltpu.sync_copy(tmp_v, counts_sh.at[pl.ds(s * _c(RSL), RSL)])
    plsc.subcore_barrier()

    def count_blk(blk, _):
        off = s * _c(2 * C32) + blk * _c(B)
        row0 = s * _c(2 * C32 // 128) + blk * _c(B // 128)
        pltpu.sync_copy(dst2d.at[pl.ds(row0, B // 128)], dstc2)
        pltpu.sync_copy(val_h.at[pl.ds(off, B)], valc)
        descs = []
        for j in range(B // 128):
            descs.append(pltpu.async_copy(
                valc.at[pl.ds(j * 128, 128)],
                counts_sh.at[dstc2.at[_c(j)]], sem, add=True))
        for dsc in descs:
            dsc.wait()
        return 0
    lax.fori_loop(_c(0), _c(2 * NB32), count_blk, 0)
    plsc.subcore_barrier()

    pltpu.sync_copy(counts_sh.at[pl.ds(s * _c(RSL), RSL)], tmp_v)

    def rsq(i, _):
        deg = jnp.maximum(tmp_v[pl.ds(i * _c(LN), LN)], 1.0)
        tmp_v[pl.ds(i * _c(LN), LN)] = _rsqrt16(deg)
        return 0
    lax.fori_loop(_c(0), _c(RSL // LN), rsq, 0)
    pltpu.sync_copy(tmp_v, counts_sh.at[pl.ds(s * _c(RSL), RSL)])
    plsc.subcore_barrier()

    pltpu.sync_copy(counts_sh, rd_full)

    wid = s * _c(NC) + c
    it = jnp.arange(LN, dtype=_i32)
    nsrc = (it * _c(641)) & _c(16383)
    ndst = (it * _c(389)) & _c(8191)

    def comp_blk(blk, carry):
        h0, h1 = carry
        off = wid * _c(C32) + blk * _c(B)
        row0 = wid * _c(C32 // 128) + blk * _c(B // 128)
        pltpu.sync_copy(src2d.at[pl.ds(row0, B // 128)], srcc2)
        pltpu.sync_copy(dst2d.at[pl.ds(row0, B // 128)], dstc2)
        pltpu.sync_copy(msk_h.at[pl.ds(off, B)], mskc)

        def vreg(i, oo):
            o0, o1 = oo
            j = i >> _c(3)
            kc = (i & _c(7)) * _c(LN)
            sv = srcc2[j, pl.ds(kc, LN)]
            dv = dstc2[j, pl.ds(kc, LN)]
            m = mskc[pl.ds(i * _c(LN), LN)]
            rs = plsc.load_gather(rd_full, [sv])
            rdd = plsc.load_gather(rd_full, [dv])
            nrm = m * rs * rdd
            g = jnp.where(sv >= _c(HALF), sv + _c(GAP), sv)
            m0 = dv < _c(HALF)
            plsc.store_compressed(bs0.at[pl.ds(o0, LN)], g, mask=m0)
            plsc.store_compressed(bd0.at[pl.ds(o0, LN)], dv, mask=m0)
            plsc.store_compressed(bn0.at[pl.ds(o0, LN)], nrm, mask=m0)
            m1 = jnp.logical_not(m0)
            plsc.store_compressed(bs1.at[pl.ds(o1, LN)], g, mask=m1)
            plsc.store_compressed(bd1.at[pl.ds(o1, LN)], dv - _c(HALF),
                                  mask=m1)
            plsc.store_compressed(bn1.at[pl.ds(o1, LN)], nrm, mask=m1)
            c0 = plsc.all_reduce_population_count(m0)[0]
            return (o0 + c0, o1 + (_c(LN) - c0))
        o0, o1 = lax.fori_loop(_c(0), _c(B // LN), vreg, (_c(0), _c(0)))

        bs0[pl.ds(o0, LN)] = nsrc
        bd0[pl.ds(o0, LN)] = ndst
        bn0[pl.ds(o0, LN)] = jnp.zeros((LN,), _f32)
        bs1[pl.ds(o1, LN)] = nsrc
        bd1[pl.ds(o1, LN)] = ndst
        bn1[pl.ds(o1, LN)] = jnp.zeros((LN,), _f32)
        ha = pl.multiple_of(h0, 8)
        hb = pl.multiple_of(h1, 8)
        pltpu.sync_copy(bs0, srcc_h.at[_c(0), wid, pl.ds(ha, FB)])
        pltpu.sync_copy(bd0, dstc_h.at[_c(0), wid, pl.ds(ha, FB)])
        pltpu.sync_copy(bn0, nrmc_h.at[_c(0), wid, pl.ds(ha, FB)])
        pltpu.sync_copy(bs1, srcc_h.at[_c(1), wid, pl.ds(hb, FB)])
        pltpu.sync_copy(bd1, dstc_h.at[_c(1), wid, pl.ds(hb, FB)])
        pltpu.sync_copy(bn1, nrmc_h.at[_c(1), wid, pl.ds(hb, FB)])
        return (h0 + ((o0 + _c(7)) & _c(-8)), h1 + ((o1 + _c(7)) & _c(-8)))
    h0, h1 = lax.fori_loop(_c(0), _c(NB32), comp_blk, (_c(0), _c(0)))

    def nfill(t, _):
        o = t * _c(LN)
        bs0[pl.ds(o, LN)] = nsrc
        bd0[pl.ds(o, LN)] = ndst
        bn0[pl.ds(o, LN)] = jnp.zeros((LN,), _f32)
        return 0
    lax.fori_loop(_c(0), _c(FB // LN), nfill, 0)
    ha = pl.multiple_of(h0, 8)
    hb = pl.multiple_of(h1, 8)
    pltpu.sync_copy(bs0, srcc_h.at[_c(0), wid, pl.ds(ha, FB)])
    pltpu.sync_copy(bd0, dstc_h.at[_c(0), wid, pl.ds(ha, FB)])
    pltpu.sync_copy(bn0, nrmc_h.at[_c(0), wid, pl.ds(ha, FB)])
    pltpu.sync_copy(bs0, srcc_h.at[_c(1), wid, pl.ds(hb, FB)])
    pltpu.sync_copy(bd0, dstc_h.at[_c(1), wid, pl.ds(hb, FB)])
    pltpu.sync_copy(bn0, nrmc_h.at[_c(1), wid, pl.ds(hb, FB)])

    crow = jnp.where(it == _c(0), h0, jnp.where(it == _c(1), h1, _c(0)))
    cvec[pl.ds(_c(0), LN)] = crow
    pltpu.sync_copy(cvec, cnts_h.at[wid])


def _edge_pass(c, s, srcc4, dstc4, nrmc3, cnts_h, emb_h,
               srcc2, dstc2, nrmc, rows_v, cv, acc_sh, sem,
               sg0, sg1, sg2, ss0, ss1, ss2):
    zv = jnp.zeros((LN,), _f32)

    def zb(i, _):
        rows_v[i >> _c(2), pl.ds((i & _c(3)) * _c(LN), LN)] = zv
        return 0
    lax.fori_loop(_c(0), _c(ZR * D // LN), zb, 0)
    for k in range(TROWS // ZR):
        pltpu.sync_copy(rows_v.at[pl.ds(0, ZR)],
                        acc_sh.at[pl.ds(s * _c(TROWS) + _c(k * ZR), ZR)])
    plsc.subcore_barrier()

    pltpu.sync_copy(cnts_h.at[pl.ds(s * _c(2), 2)], cv)
    v0 = cv[_c(0), pl.ds(0, LN)]
    v1 = cv[_c(1), pl.ds(0, LN)]
    is0 = c == _c(0)
    nn = (jnp.where(is0, v0[0], v0[1]), jnp.where(is0, v1[0], v1[1]))

    for li in range(2):
        lt = s * _c(2) + _c(li)
        nblk = (nn[li] + _c(B - 1)) >> _c(10)

        def blk_fn(blk, _):
            pltpu.sync_copy(srcc4.at[c, lt, pl.ds(blk * _c(B // 128),
                                                  B // 128)], srcc2)
            pltpu.sync_copy(dstc4.at[c, lt, pl.ds(blk * _c(B // 128),
                                                  B // 128)], dstc2)
            pltpu.sync_copy(nrmc3.at[c, lt, pl.ds(blk * _c(B), B)], nrmc)

            slots = [rows_v.at[pl.ds(128 * t, 128)] for t in range(3)]
            sgs = (sg0, sg1, sg2)
            sss = (ss0, ss1, ss2)

            def g_desc(qq, t):
                return pltpu.make_async_copy(
                    emb_h.at[srcc2.at[_c(qq)]], slots[t], sgs[t])

            def s_desc(qq, t):
                return pltpu.make_async_copy(
                    slots[t], acc_sh.at[dstc2.at[_c(qq)]], sss[t])

            def scale_slot(t, q):
                @plsc.parallel_loop(_c(0), _c(128 // LN), _c(1), unroll=2)
                def scale(g):
                    w16 = nrmc[pl.ds(_c(q * 128) + g * _c(LN), LN)]
                    e0 = _c(t * 128) + g * _c(LN)
                    for l in range(LN):
                        e = e0 + _c(l)
                        w = w16[jnp.full((LN,), l, _i32)]
                        for kk in range(D // LN):
                            rows_v[e, pl.ds(kk * LN, LN)] = (
                                rows_v[e, pl.ds(kk * LN, LN)] * w)

            nq = B // SB
            g_desc(0, 0).start()
            g_desc(1, 1).start()
            for q in range(nq):
                t = q % 3
                g_desc(q, t).wait()
                scale_slot(t, q)
                s_desc(q, t).start(add=True)
                if q + 2 < nq:
                    if q - 1 >= 0:
                        s_desc(q - 1, (q + 2) % 3).wait()
                    g_desc(q + 2, (q + 2) % 3).start()
            for q in range(nq - 3, nq):
                s_desc(q, q % 3).wait()
            return 0
        lax.fori_loop(_c(0), nblk, blk_fn, 0)
    plsc.subcore_barrier()


def _layer_body(emb_h, srcc4, dstc4, nrmc3, cnts_h, out_h,
                srcc2, dstc2, nrmc, rows_v, cv, acc_sh, sem,
                sg0, sg1, sg2, ss0, ss1, ss2):
    c = lax.axis_index("c")
    s = lax.axis_index("s")
    _edge_pass(c, s, srcc4, dstc4, nrmc3, cnts_h, emb_h,
               srcc2, dstc2, nrmc, rows_v, cv, acc_sh, sem,
               sg0, sg1, sg2, ss0, ss1, ss2)
    pltpu.sync_copy(acc_sh.at[pl.ds(s * _c(TROWS), TROWS)],
                    out_h.at[pl.ds(c * _c(HCAP) + s * _c(TROWS), TROWS)])


def _final_body(emb_h, srcc4, dstc4, nrmc3, cnts_h, e0p_h, e1p_h, out_h,
                srcc2, dstc2, nrmc, rows_v, cv, acc_sh, sem,
                sg0, sg1, sg2, ss0, ss1, ss2):
    c = lax.axis_index("c")
    s = lax.axis_index("s")
    _edge_pass(c, s, srcc4, dstc4, nrmc3, cnts_h, emb_h,
               srcc2, dstc2, nrmc, rows_v, cv, acc_sh, sem,
               sg0, sg1, sg2, ss0, ss1, ss2)

    start_pad = c * _c(HCAP) + s * _c(TROWS)
    is_edge = s == _c(NS - 1)

    def mean_blk(k, _):
        boff = k * _c(ZR)
        boff = jnp.where(is_edge, jnp.minimum(boff, _c(TROWS - GAP - ZR)),
                         boff)
        row_pad = start_pad + boff
        out_row = row_pad - _c(GAP) * c
        d0 = pltpu.async_copy(acc_sh.at[pl.ds(s * _c(TROWS) + boff, ZR)],
                              rows_v.at[pl.ds(0, ZR)], sg0)
        d1 = pltpu.async_copy(e0p_h.at[pl.ds(row_pad, ZR)],
                              rows_v.at[pl.ds(ZR, ZR)], sg1)
        d2 = pltpu.async_copy(e1p_h.at[pl.ds(row_pad, ZR)],
                              rows_v.at[pl.ds(2 * ZR, ZR)], ss0)
        d3 = pltpu.async_copy(emb_h.at[pl.ds(row_pad, ZR)],
                              rows_v.at[pl.ds(3 * ZR, ZR)], ss1)
        for dd in (d0, d1, d2, d3):
            dd.wait()

        def vreg(i, _):
            r = i >> _c(2)
            kc = (i & _c(3)) * _c(LN)
            v = (rows_v[r, pl.ds(kc, LN)]
                 + rows_v[_c(ZR) + r, pl.ds(kc, LN)]
                 + rows_v[_c(2 * ZR) + r, pl.ds(kc, LN)]
                 + rows_v[_c(3 * ZR) + r, pl.ds(kc, LN)])
            rows_v[r, pl.ds(kc, LN)] = v * 0.25
            return 0
        lax.fori_loop(_c(0), _c(ZR * D // LN), vreg, 0)
        pltpu.sync_copy(rows_v.at[pl.ds(0, ZR)],
                        out_h.at[pl.ds(out_row, ZR)])
        return 0
    lax.fori_loop(_c(0), _c(TROWS // ZR), mean_blk, 0)


@functools.lru_cache(maxsize=1)
def _get_calls():
    mesh = plsc.VectorSubcoreMesh(core_axis_name="c", subcore_axis_name="s",
                                  num_cores=NC, num_subcores=NS)
    params = pltpu.CompilerParams(needs_layout_passes=False,
                                  use_tc_tiling_on_sc=False)
    prep_call = pl.kernel(
        _prep_body,
        out_type=(
            jax.ShapeDtypeStruct((NC, 32, CAP), _i32),
            jax.ShapeDtypeStruct((NC, 32, CAP), _i32),
            jax.ShapeDtypeStruct((NC, 32, CAP), _f32),
            jax.ShapeDtypeStruct((32, LN), _i32),
        ),
        mesh=mesh,
        compiler_params=params,
        scratch_types=[
            pltpu.VMEM((8, 128), _i32),
            pltpu.VMEM((8, 128), _i32),
            pltpu.VMEM((B,), _f32),
            pltpu.VMEM((B,), _f32),
            pltpu.VMEM((RSL,), _f32),
            pltpu.VMEM((RDLEN,), _f32),
            pltpu.VMEM((FB,), _i32),
            pltpu.VMEM((FB,), _i32),
            pltpu.VMEM((FB,), _f32),
            pltpu.VMEM((FB,), _i32),
            pltpu.VMEM((FB,), _i32),
            pltpu.VMEM((FB,), _f32),
            pltpu.VMEM((LN,), _i32),
            pltpu.VMEM_SHARED((RDLEN,), _f32),
            pltpu.SemaphoreType.DMA,
        ],
    )
    layer_scratch = [
        pltpu.VMEM((8, 128), _i32),
        pltpu.VMEM((8, 128), _i32),
        pltpu.VMEM((B,), _f32),
        pltpu.VMEM((3 * SB, D), _f32),
        pltpu.VMEM((2, LN), _i32),
    ]
    layer_call = pl.kernel(
        _layer_body,
        out_type=jax.ShapeDtypeStruct((NPAD, D), _f32),
        mesh=mesh,
        compiler_params=params,
        scratch_types=layer_scratch + [
            pltpu.VMEM_SHARED((HCAP, D), _f32),
            pltpu.SemaphoreType.DMA,
            pltpu.SemaphoreType.DMA,
            pltpu.SemaphoreType.DMA,
            pltpu.SemaphoreType.DMA,
            pltpu.SemaphoreType.DMA,
            pltpu.SemaphoreType.DMA,
            pltpu.SemaphoreType.DMA,
        ],
    )
    final_call = pl.kernel(
        _final_body,
        out_type=jax.ShapeDtypeStruct((NN, D), _f32),
        mesh=mesh,
        compiler_params=params,
        scratch_types=layer_scratch + [
            pltpu.VMEM_SHARED((HCAP, D), _f32),
            pltpu.SemaphoreType.DMA,
            pltpu.SemaphoreType.DMA,
            pltpu.SemaphoreType.DMA,
            pltpu.SemaphoreType.DMA,
            pltpu.SemaphoreType.DMA,
            pltpu.SemaphoreType.DMA,
            pltpu.SemaphoreType.DMA,
        ],
    )
    return prep_call, layer_call, final_call


def kernel(user_emb, item_emb, edge_index, mask):
    src = edge_index[0].astype(_i32)
    dst = edge_index[1].astype(_i32)
    mask = mask.astype(_f32)

    padn = EPAD - E
    padidx = (jnp.arange(padn, dtype=_i32) * 37) % NN
    srcp = jnp.concatenate([src, padidx])
    dstp = jnp.concatenate([dst, padidx])
    maskp = jnp.concatenate([mask, jnp.zeros((padn,), _f32)])
    validp = jnp.concatenate([jnp.ones((E,), _f32), jnp.zeros((padn,), _f32)])
    src2d = srcp.reshape(EPAD // 128, 128)
    dst2d = dstp.reshape(EPAD // 128, 128)

    gap = jnp.zeros((GAP, D), _f32)
    e0p = jnp.concatenate([user_emb, item_emb[:HALF - N_USERS], gap,
                           item_emb[HALF - N_USERS:], gap])

    prep_call, layer_call, final_call = _get_calls()
    srcc, dstc, nrmc, cnts = prep_call(src2d, dst2d, maskp, validp)
    srcc4 = srcc.reshape(NC, 32, CAP // 128, 128)
    dstc4 = dstc.reshape(NC, 32, CAP // 128, 128)
    e1p = layer_call(e0p, srcc4, dstc4, nrmc, cnts)
    e2p = layer_call(e1p, srcc4, dstc4, nrmc, cnts)
    out = final_call(e2p, srcc4, dstc4, nrmc, cnts, e0p, e1p)
    return out
